# baseline (speedup 1.0000x reference)
import functools

import jax
import jax.numpy as jnp
from jax import lax
from jax.experimental import pallas as pl
from jax.experimental.pallas import tpu as pltpu
from jax.experimental.pallas import tpu_sc as plsc

B = 16384
SEQ = 10
EMB = 32
HID = 128
N_ACT = 32
NPAT = 1 << SEQ

NC = 2
NS = 16
L = 16
NW = NC * NS
RPW = B // NW
NG = RPW // L


def _lut_body(tok_ref, pos_ref, wb_ref, bb_ref, wa_ref, ba_ref, wc_ref,
              bc_ref, lut_ref, cvec_ref):
    e = tok_ref[1:2, :] + pos_ref[...]
    h = jnp.maximum(
        jnp.dot(e, wb_ref[...], preferred_element_type=jnp.float32)
        + bb_ref[...], 0.0)
    a1 = jnp.dot(h, wa_ref[...], preferred_element_type=jnp.float32)
    c1 = jnp.dot(h, wc_ref[...], preferred_element_type=jnp.float32)
    p_io = lax.broadcasted_iota(jnp.int32, (NPAT, SEQ), 0)
    s_io = lax.broadcasted_iota(jnp.int32, (NPAT, SEQ), 1)
    bits = ((p_io >> s_io) & 1).astype(jnp.float32)
    cnt = bits.sum(axis=1, keepdims=True)
    scale = 1.0 / (cnt + 1e-6)
    lut_ref[...] = (jnp.dot(bits, a1, preferred_element_type=jnp.float32)
                    * scale + ba_ref[...])
    cvec_ref[...] = jnp.concatenate(
        [c1, bc_ref[...], jnp.zeros((L - SEQ - 1, 1), jnp.float32)], axis=0)


_lut_call = pl.pallas_call(
    _lut_body,
    out_shape=[
        jax.ShapeDtypeStruct((NPAT, N_ACT), jnp.float32),
        jax.ShapeDtypeStruct((L, 1), jnp.float32),
    ],
)


def _sc_body(inpt_hbm, maskw_hbm, lut_hbm, cvec_hbm, actor_hbm, critic_hbm,
             inp_v, lut_v, out_v, maskw_v, cvec_v, crit_v, pat_v, lut_sem):
    wid = lax.axis_index("s") * NC + lax.axis_index("c")
    base = wid * RPW
    with jax.named_scope("in_copies"):
        lut_cp = pltpu.async_copy(lut_hbm, lut_v, lut_sem)
        pltpu.sync_copy(inpt_hbm.at[pl.ds(wid * SEQ * RPW, SEQ * RPW)], inp_v)
        pltpu.sync_copy(maskw_hbm.at[pl.ds(base, RPW)], maskw_v)
        pltpu.sync_copy(cvec_hbm, cvec_v)

    cv = cvec_v[pl.ds(0, L)]
    c1 = [cv[s] for s in range(SEQ)]
    bc = cv[SEQ]
    iota = lax.iota(jnp.int32, L)
    neg = jnp.full((L,), -1e9, jnp.float32)

    def _pat(g):
        r0 = g * L
        p = jnp.zeros((L,), jnp.int32)
        acc = jnp.zeros((L,), jnp.float32)
        cnt = jnp.zeros((L,), jnp.float32)
        one = jnp.full((L,), 1.0, jnp.float32)
        zero = jnp.zeros((L,), jnp.float32)
        for s in range(SEQ):
            v = inp_v[pl.ds(s * RPW + r0, L)]
            m = v == 1
            p = p | jnp.where(m, jnp.int32(1 << s), jnp.int32(0))
            ind = jnp.where(m, one, zero)
            acc = acc + ind * c1[s]
            cnt = cnt + ind
        crit_v[pl.ds(r0, L)] = acc / (cnt + 1e-6) + bc
        pat_v[pl.ds(r0, L)] = p * N_ACT
    with jax.named_scope("pattern_pass"):
        plsc.parallel_loop(0, NG)(_pat)

    with jax.named_scope("lut_wait"):
        lut_cp.wait()

    def _gth(g):
        r0 = g * L
        rvec = r0 + iota
        mw = maskw_v[pl.ds(r0, L)]
        pbase = pat_v[pl.ds(r0, L)]
        rbase = rvec * N_ACT
        for j in range(N_ACT):
            cids = (iota + j) & (N_ACT - 1)
            lg = plsc.load_gather(lut_v, [pbase + cids])
            mk = (mw >> cids) & 1
            out = jnp.where(mk != 0, lg, neg)
            plsc.store_scatter(out_v, [rbase + cids], out)
    with jax.named_scope("gather_pass"):
        plsc.parallel_loop(0, NG)(_gth)

    with jax.named_scope("out_copies"):
        pltpu.sync_copy(out_v, actor_hbm.at[pl.ds(base * N_ACT, RPW * N_ACT)])
        pltpu.sync_copy(crit_v, critic_hbm.at[pl.ds(base, RPW)])


@functools.lru_cache(maxsize=1)
def _get_sc_call():
    mesh = plsc.VectorSubcoreMesh(core_axis_name="c", subcore_axis_name="s")
    return pl.kernel(
        _sc_body,
        mesh=mesh,
        compiler_params=pltpu.CompilerParams(needs_layout_passes=False),
        out_type=[
            jax.ShapeDtypeStruct((B * N_ACT,), jnp.float32),
            jax.ShapeDtypeStruct((B,), jnp.float32),
        ],
        scratch_types=[
            pltpu.VMEM((SEQ * RPW,), jnp.int32),
            pltpu.VMEM((NPAT * N_ACT,), jnp.float32),
            pltpu.VMEM((RPW * N_ACT,), jnp.float32),
            pltpu.VMEM((RPW,), jnp.int32),
            pltpu.VMEM((L,), jnp.float32),
            pltpu.VMEM((RPW,), jnp.float32),
            pltpu.VMEM((RPW,), jnp.int32),
            pltpu.SemaphoreType.DMA,
        ],
    )


def kernel(inputs, action_mask, token_table, pos_table, W_body, b_body,
           W_actor, b_actor, W_critic, b_critic):
    lut, cvec = _lut_call(
        token_table, pos_table, W_body, b_body.reshape(1, HID),
        W_actor, b_actor.reshape(1, N_ACT), W_critic, b_critic.reshape(1, 1))
    maskw = jnp.sum(
        action_mask.astype(jnp.int32)
        << jnp.arange(N_ACT, dtype=jnp.int32)[None, :], axis=1)
    inpt = inputs.reshape(NW, RPW, SEQ).transpose(0, 2, 1).reshape(-1)
    actor, critic = _get_sc_call()(
        inpt, maskw, lut.reshape(-1), cvec.reshape(L))
    return actor.reshape(B, N_ACT), critic.reshape(B, 1)


# speedup vs baseline: 1.0965x; 1.0965x over previous
import functools

import jax
import jax.numpy as jnp
from jax import lax
from jax.experimental import pallas as pl
from jax.experimental.pallas import tpu as pltpu
from jax.experimental.pallas import tpu_sc as plsc

B = 16384
SEQ = 10
EMB = 32
HID = 128
N_ACT = 32
NPAT = 1 << SEQ

NC = 2
NS = 16
L = 16
NW = NC * NS
RPW = B // NW
NG = RPW // L


def _lut_body(tok_ref, pos_ref, wb_ref, bb_ref, wa_ref, ba_ref, wc_ref,
              bc_ref, lut_ref, cvec_ref):
    e = tok_ref[1:2, :] + pos_ref[...]
    h = jnp.maximum(
        jnp.dot(e, wb_ref[...], preferred_element_type=jnp.float32)
        + bb_ref[...], 0.0)
    a1 = jnp.dot(h, wa_ref[...], preferred_element_type=jnp.float32)
    c1 = jnp.dot(h, wc_ref[...], preferred_element_type=jnp.float32)
    p_io = lax.broadcasted_iota(jnp.int32, (NPAT, SEQ), 0)
    s_io = lax.broadcasted_iota(jnp.int32, (NPAT, SEQ), 1)
    bits = ((p_io >> s_io) & 1).astype(jnp.float32)
    cnt = bits.sum(axis=1, keepdims=True)
    scale = 1.0 / (cnt + 1e-6)
    lut_ref[...] = (jnp.dot(bits, a1, preferred_element_type=jnp.float32)
                    * scale + ba_ref[...])
    cvec_ref[...] = jnp.concatenate(
        [c1, bc_ref[...], jnp.zeros((L - SEQ - 1, 1), jnp.float32)], axis=0)


_lut_call = pl.pallas_call(
    _lut_body,
    out_shape=[
        jax.ShapeDtypeStruct((NPAT, N_ACT), jnp.float32),
        jax.ShapeDtypeStruct((L, 1), jnp.float32),
    ],
)


def _sc_body(inpt_hbm, maskw_hbm, lut_hbm, cvec_hbm, actor_hbm, critic_hbm,
             inp_v, lut_v, out_v, maskw_v, cvec_v, crit_v, pat_v, lut_sem):
    wid = lax.axis_index("s") * NC + lax.axis_index("c")
    base = wid * RPW
    with jax.named_scope("in_copies"):
        lut_cp = pltpu.async_copy(lut_hbm, lut_v, lut_sem)
        pltpu.sync_copy(inpt_hbm.at[pl.ds(wid * SEQ * RPW, SEQ * RPW)], inp_v)
        pltpu.sync_copy(maskw_hbm.at[pl.ds(base, RPW)], maskw_v)
        pltpu.sync_copy(cvec_hbm, cvec_v)

    cv = cvec_v[pl.ds(0, L)]
    c1 = [cv[s] for s in range(SEQ)]
    bc = cv[SEQ]
    iota = lax.iota(jnp.int32, L)
    neg = jnp.full((L,), -1e9, jnp.float32)

    def _pat(g):
        r0 = g * L
        p = jnp.zeros((L,), jnp.int32)
        acc = jnp.zeros((L,), jnp.float32)
        cnt = jnp.zeros((L,), jnp.float32)
        one = jnp.full((L,), 1.0, jnp.float32)
        zero = jnp.zeros((L,), jnp.float32)
        for s in range(SEQ):
            v = inp_v[pl.ds(s * RPW + r0, L)]
            m = v == 1
            p = p | jnp.where(m, jnp.int32(1 << s), jnp.int32(0))
            ind = jnp.where(m, one, zero)
            acc = acc + ind * c1[s]
            cnt = cnt + ind
        crit_v[pl.ds(r0, L)] = acc / (cnt + 1e-6) + bc
        pat_v[pl.ds(r0, L)] = p * N_ACT
    with jax.named_scope("pattern_pass"):
        plsc.parallel_loop(0, NG)(_pat)

    with jax.named_scope("lut_wait"):
        lut_cp.wait()

    def _gth(g):
        r0 = g * L
        rvec = r0 + iota
        mw = maskw_v[pl.ds(r0, L)]
        pbase = pat_v[pl.ds(r0, L)]
        for j in range(N_ACT):
            cids = (iota + j) & (N_ACT - 1)
            lg = plsc.load_gather(lut_v, [pbase + cids])
            mk = (mw >> cids) & 1
            out = jnp.where(mk != 0, lg, neg)
            plsc.store_scatter(out_v, [rvec, cids], out)
    with jax.named_scope("gather_pass"):
        plsc.parallel_loop(0, NG)(_gth)

    with jax.named_scope("out_copies"):
        pltpu.sync_copy(out_v, actor_hbm.at[pl.ds(base, RPW), :])
        pltpu.sync_copy(crit_v, critic_hbm.at[pl.ds(base, RPW)])


@functools.lru_cache(maxsize=1)
def _get_sc_call():
    mesh = plsc.VectorSubcoreMesh(core_axis_name="c", subcore_axis_name="s")
    return pl.kernel(
        _sc_body,
        mesh=mesh,
        compiler_params=pltpu.CompilerParams(needs_layout_passes=False),
        out_type=[
            jax.ShapeDtypeStruct((B, N_ACT), jnp.float32),
            jax.ShapeDtypeStruct((B,), jnp.float32),
        ],
        scratch_types=[
            pltpu.VMEM((SEQ * RPW,), jnp.int32),
            pltpu.VMEM((NPAT * N_ACT,), jnp.float32),
            pltpu.VMEM((RPW, N_ACT), jnp.float32),
            pltpu.VMEM((RPW,), jnp.int32),
            pltpu.VMEM((L,), jnp.float32),
            pltpu.VMEM((RPW,), jnp.float32),
            pltpu.VMEM((RPW,), jnp.int32),
            pltpu.SemaphoreType.DMA,
        ],
    )


def kernel(inputs, action_mask, token_table, pos_table, W_body, b_body,
           W_actor, b_actor, W_critic, b_critic):
    lut, cvec = _lut_call(
        token_table, pos_table, W_body, b_body.reshape(1, HID),
        W_actor, b_actor.reshape(1, N_ACT), W_critic, b_critic.reshape(1, 1))
    maskw = jnp.sum(
        action_mask.astype(jnp.int32)
        << jnp.arange(N_ACT, dtype=jnp.int32)[None, :], axis=1)
    inpt = inputs.reshape(NW, RPW, SEQ).transpose(0, 2, 1).reshape(-1)
    actor, critic = _get_sc_call()(
        inpt, maskw, lut.reshape(-1), cvec.reshape(L))
    return actor, critic.reshape(B, 1)


# speedup vs baseline: 1.1052x; 1.0079x over previous
import functools

import jax
import jax.numpy as jnp
from jax import lax
from jax.experimental import pallas as pl
from jax.experimental.pallas import tpu as pltpu
from jax.experimental.pallas import tpu_sc as plsc

B = 16384
SEQ = 10
EMB = 32
HID = 128
N_ACT = 32
NPAT = 1 << SEQ

NC = 2
NS = 16
L = 16
NW = NC * NS
RPW = B // NW
NG = RPW // L


def _lut_body(tok_ref, pos_ref, wb_ref, bb_ref, wa_ref, ba_ref, wc_ref,
              bc_ref, lut_ref, cvec_ref):
    e = tok_ref[1:2, :] + pos_ref[...]
    h = jnp.maximum(
        jnp.dot(e, wb_ref[...], preferred_element_type=jnp.float32)
        + bb_ref[...], 0.0)
    a1 = jnp.dot(h, wa_ref[...], preferred_element_type=jnp.float32)
    c1 = jnp.dot(h, wc_ref[...], preferred_element_type=jnp.float32)
    p_io = lax.broadcasted_iota(jnp.int32, (NPAT, SEQ), 0)
    s_io = lax.broadcasted_iota(jnp.int32, (NPAT, SEQ), 1)
    bits = ((p_io >> s_io) & 1).astype(jnp.float32)
    cnt = bits.sum(axis=1, keepdims=True)
    scale = 1.0 / (cnt + 1e-6)
    lut_ref[...] = (jnp.dot(bits, a1, preferred_element_type=jnp.float32)
                    * scale + ba_ref[...])
    cvec_ref[...] = jnp.concatenate(
        [c1, bc_ref[...], jnp.zeros((L - SEQ - 1, 1), jnp.float32)], axis=0)


_lut_call = pl.pallas_call(
    _lut_body,
    out_shape=[
        jax.ShapeDtypeStruct((NPAT, N_ACT), jnp.float32),
        jax.ShapeDtypeStruct((L, 1), jnp.float32),
    ],
)


def _sc_body(inpt_hbm, maskw_hbm, lut_hbm, cvec_hbm, actor_hbm, critic_hbm,
             inp_v, lut_v, out_v, maskw_v, cvec_v, crit_v, pat_v, lut_sem,
             mask_sem):
    wid = lax.axis_index("s") * NC + lax.axis_index("c")
    base = wid * RPW
    with jax.named_scope("in_copies"):
        lut_cp = pltpu.async_copy(lut_hbm, lut_v, lut_sem)
        mask_cp = pltpu.async_copy(
            maskw_hbm.at[pl.ds(base, RPW)], maskw_v, mask_sem)
        pltpu.sync_copy(inpt_hbm.at[pl.ds(wid * SEQ * RPW, SEQ * RPW)], inp_v)
        pltpu.sync_copy(cvec_hbm, cvec_v)

    cv = cvec_v[pl.ds(0, L)]
    c1 = [cv[s] for s in range(SEQ)]
    bc = cv[SEQ]
    iota = lax.iota(jnp.int32, L)
    neg = jnp.full((L,), -1e9, jnp.float32)

    def _pat(g):
        r0 = g * L
        p = jnp.zeros((L,), jnp.int32)
        acc = jnp.zeros((L,), jnp.float32)
        cnt = jnp.zeros((L,), jnp.float32)
        one = jnp.full((L,), 1.0, jnp.float32)
        zero = jnp.zeros((L,), jnp.float32)
        for s in range(SEQ):
            v = inp_v[pl.ds(s * RPW + r0, L)]
            m = v == 1
            p = p | jnp.where(m, jnp.int32(1 << s), jnp.int32(0))
            ind = jnp.where(m, one, zero)
            acc = acc + ind * c1[s]
            cnt = cnt + ind
        crit_v[pl.ds(r0, L)] = acc / (cnt + 1e-6) + bc
        pat_v[pl.ds(r0, L)] = p * N_ACT
    with jax.named_scope("pattern_pass"):
        plsc.parallel_loop(0, NG)(_pat)

    with jax.named_scope("lut_wait"):
        lut_cp.wait()
        mask_cp.wait()

    def _gth(g):
        r0 = g * L
        rvec = r0 + iota
        mw = maskw_v[pl.ds(r0, L)]
        pbase = pat_v[pl.ds(r0, L)]
        for j in range(N_ACT):
            cids = (iota + j) & (N_ACT - 1)
            lg = plsc.load_gather(lut_v, [pbase + cids])
            mk = (mw >> cids) & 1
            out = jnp.where(mk != 0, lg, neg)
            plsc.store_scatter(out_v, [rvec, cids], out)
    with jax.named_scope("gather_pass"):
        plsc.parallel_loop(0, NG)(_gth)

    with jax.named_scope("out_copies"):
        pltpu.sync_copy(out_v, actor_hbm.at[pl.ds(base, RPW), :])
        pltpu.sync_copy(crit_v, critic_hbm.at[pl.ds(base, RPW)])


@functools.lru_cache(maxsize=1)
def _get_sc_call():
    mesh = plsc.VectorSubcoreMesh(core_axis_name="c", subcore_axis_name="s")
    return pl.kernel(
        _sc_body,
        mesh=mesh,
        compiler_params=pltpu.CompilerParams(needs_layout_passes=False),
        out_type=[
            jax.ShapeDtypeStruct((B, N_ACT), jnp.float32),
            jax.ShapeDtypeStruct((B,), jnp.float32),
        ],
        scratch_types=[
            pltpu.VMEM((SEQ * RPW,), jnp.int32),
            pltpu.VMEM((NPAT * N_ACT,), jnp.float32),
            pltpu.VMEM((RPW, N_ACT), jnp.float32),
            pltpu.VMEM((RPW,), jnp.int32),
            pltpu.VMEM((L,), jnp.float32),
            pltpu.VMEM((RPW,), jnp.float32),
            pltpu.VMEM((RPW,), jnp.int32),
            pltpu.SemaphoreType.DMA,
            pltpu.SemaphoreType.DMA,
        ],
    )


def kernel(inputs, action_mask, token_table, pos_table, W_body, b_body,
           W_actor, b_actor, W_critic, b_critic):
    lut, cvec = _lut_call(
        token_table, pos_table, W_body, b_body.reshape(1, HID),
        W_actor, b_actor.reshape(1, N_ACT), W_critic, b_critic.reshape(1, 1))
    maskw = jnp.sum(
        action_mask.astype(jnp.int32)
        << jnp.arange(N_ACT, dtype=jnp.int32)[None, :], axis=1)
    inpt = inputs.reshape(NW, RPW, SEQ).transpose(0, 2, 1).reshape(-1)
    actor, critic = _get_sc_call()(
        inpt, maskw, lut.reshape(-1), cvec.reshape(L))
    return actor, critic.reshape(B, 1)
